# baseline (speedup 1.0000x reference)
import functools

import jax
import jax.numpy as jnp
from jax import lax
from jax.experimental import pallas as pl
from jax.experimental.pallas import tpu as pltpu
from jax.experimental.pallas import tpu_sc as plsc

B, N, F, D = 64, 256, 9, 768
NC, NS, L = 2, 16, 16
LB = 2 * L
NW = NC * NS
R = B * N
RPW = R // NW
C = 8
NCHUNK = RPW // C
OUT_ROWS = B * (N + 1)

_mesh = plsc.VectorSubcoreMesh(core_axis_name="c", subcore_axis_name="s")


@functools.partial(
    pl.kernel,
    out_type=jax.ShapeDtypeStruct((OUT_ROWS, D), jnp.bfloat16),
    mesh=_mesh,
    compiler_params=pltpu.CompilerParams(use_tc_tiling_on_sc=False),
    scratch_types=[
        pltpu.VMEM((NCHUNK, C * F), jnp.int32),
        pltpu.VMEM((NCHUNK, C), jnp.int32),
        pltpu.VMEM((NCHUNK, C), jnp.int32),
        pltpu.VMEM((2, C * F, D), jnp.bfloat16),
        pltpu.VMEM((2, C, D), jnp.bfloat16),
        pltpu.VMEM((2, C, D), jnp.bfloat16),
        pltpu.VMEM((2, C, D), jnp.bfloat16),
        pltpu.VMEM((1, D), jnp.bfloat16),
        pltpu.SemaphoreType.DMA,
        pltpu.SemaphoreType.DMA,
        pltpu.SemaphoreType.DMA,
        pltpu.SemaphoreType.DMA,
    ],
)
def _sc_body(x_hbm, ind_hbm, outd_hbm, atab, itab, otab, tok, out_hbm,
             x_v, ind_v, outd_v, arows, irows, orows, out_v, tok_v,
             semg0, semg1, semo0, semo1):
    w = lax.axis_index("s") * NC + lax.axis_index("c")
    semg = (semg0, semg1)
    semo = (semo0, semo1)

    pltpu.sync_copy(x_hbm.at[w], x_v)
    pltpu.sync_copy(ind_hbm.at[w], ind_v)
    pltpu.sync_copy(outd_hbm.at[w], outd_v)
    pltpu.sync_copy(tok, tok_v)
    b0 = w * (RPW // N)
    for k in range(RPW // N):
        pltpu.sync_copy(tok_v, out_hbm.at[pl.ds((b0 + k) * (N + 1), 1)])

    def fire_gathers(c, p):
        pltpu.async_copy(atab.at[x_v.at[c]], arows.at[p], semg[p])
        pltpu.async_copy(itab.at[ind_v.at[c]], irows.at[p], semg[p])
        pltpu.async_copy(otab.at[outd_v.at[c]], orows.at[p], semg[p])

    def wait_gathers(c, p):
        pltpu.make_async_copy(atab.at[x_v.at[c]], arows.at[p], semg[p]).wait()
        pltpu.make_async_copy(itab.at[ind_v.at[c]], irows.at[p], semg[p]).wait()
        pltpu.make_async_copy(otab.at[outd_v.at[c]], orows.at[p], semg[p]).wait()

    def out_row(c):
        r0 = w * RPW + c * C
        return r0 + r0 // N + 1

    def out_copy(c, p):
        return pltpu.make_async_copy(
            out_v.at[p], out_hbm.at[pl.ds(out_row(c), C)], semo[p])

    fire_gathers(0, 0)

    @pl.loop(0, NCHUNK, step=2)
    def _c2(c0):
        for p in range(2):
            c = c0 + p
            q = 1 - p

            @pl.when(c + 1 < NCHUNK)
            def _():
                fire_gathers(c + 1, q)

            wait_gathers(c, p)

            @pl.when(c >= 2)
            def _():
                out_copy(c - 2, p).wait()

            @pl.loop(0, D // LB)
            def _cols(j):
                sl = pl.ds(j * LB, LB)
                for i in range(C):
                    acc = irows[p, i, sl] + orows[p, i, sl]
                    for f in range(F):
                        acc = acc + arows[p, i * F + f, sl]
                    out_v[p, i, sl] = acc

            out_copy(c, p).start()

    out_copy(NCHUNK - 2, 0).wait()
    out_copy(NCHUNK - 1, 1).wait()


def kernel(x, in_degree, out_degree, atom_table, in_deg_table, out_deg_table,
           graph_token):
    x3 = x.reshape(NW, NCHUNK, C * F)
    ind3 = in_degree.reshape(NW, NCHUNK, C)
    outd3 = out_degree.reshape(NW, NCHUNK, C)
    out = _sc_body(x3, ind3, outd3,
                   atom_table.astype(jnp.bfloat16),
                   in_deg_table.astype(jnp.bfloat16),
                   out_deg_table.astype(jnp.bfloat16),
                   graph_token.astype(jnp.bfloat16))
    return out.astype(jnp.float32).reshape(B, N + 1, D)


# speedup vs baseline: 1.1145x; 1.1145x over previous
import functools

import jax
import jax.numpy as jnp
from jax import lax
from jax.experimental import pallas as pl
from jax.experimental.pallas import tpu as pltpu
from jax.experimental.pallas import tpu_sc as plsc

B, N, F, D = 64, 256, 9, 768
NC, NS, L = 2, 16, 16
NW = NC * NS
R = B * N
RPW = R // NW
C = 4
NCHUNK = RPW // C
OUT_ROWS = B * (N + 1)

_mesh = plsc.VectorSubcoreMesh(core_axis_name="c", subcore_axis_name="s")


@functools.partial(
    pl.kernel,
    out_type=jax.ShapeDtypeStruct((OUT_ROWS, D), jnp.float32),
    mesh=_mesh,
    compiler_params=pltpu.CompilerParams(use_tc_tiling_on_sc=False),
    scratch_types=[
        pltpu.VMEM((NCHUNK, C * F), jnp.int32),
        pltpu.VMEM((NCHUNK, C), jnp.int32),
        pltpu.VMEM((NCHUNK, C), jnp.int32),
        pltpu.VMEM((2, C * F, D), jnp.float32),
        pltpu.VMEM((2, C, D), jnp.float32),
        pltpu.VMEM((2, C, D), jnp.float32),
        pltpu.VMEM((2, C, D), jnp.float32),
        pltpu.VMEM((1, D), jnp.float32),
        pltpu.SemaphoreType.DMA,
        pltpu.SemaphoreType.DMA,
        pltpu.SemaphoreType.DMA,
        pltpu.SemaphoreType.DMA,
    ],
)
def _sc_body(x_hbm, ind_hbm, outd_hbm, atab, itab, otab, tok, out_hbm,
             x_v, ind_v, outd_v, arows, irows, orows, out_v, tok_v,
             semg0, semg1, semo0, semo1):
    w = lax.axis_index("s") * NC + lax.axis_index("c")
    semg = (semg0, semg1)
    semo = (semo0, semo1)

    pltpu.sync_copy(x_hbm.at[w], x_v)
    pltpu.sync_copy(ind_hbm.at[w], ind_v)
    pltpu.sync_copy(outd_hbm.at[w], outd_v)
    pltpu.sync_copy(tok, tok_v)
    b0 = w * (RPW // N)
    for k in range(RPW // N):
        pltpu.sync_copy(tok_v, out_hbm.at[pl.ds((b0 + k) * (N + 1), 1)])

    def fire_gathers(c, p):
        pltpu.async_copy(atab.at[x_v.at[c]], arows.at[p], semg[p])
        pltpu.async_copy(itab.at[ind_v.at[c]], irows.at[p], semg[p])
        pltpu.async_copy(otab.at[outd_v.at[c]], orows.at[p], semg[p])

    def wait_gathers(c, p):
        pltpu.make_async_copy(atab.at[x_v.at[c]], arows.at[p], semg[p]).wait()
        pltpu.make_async_copy(itab.at[ind_v.at[c]], irows.at[p], semg[p]).wait()
        pltpu.make_async_copy(otab.at[outd_v.at[c]], orows.at[p], semg[p]).wait()

    def out_row(c):
        r0 = w * RPW + c * C
        return r0 + r0 // N + 1

    def out_copy(c, p):
        return pltpu.make_async_copy(
            out_v.at[p], out_hbm.at[pl.ds(out_row(c), C)], semo[p])

    fire_gathers(0, 0)

    @pl.loop(0, NCHUNK, step=2)
    def _c2(c0):
        for p in range(2):
            c = c0 + p
            q = 1 - p

            @pl.when(c + 1 < NCHUNK)
            def _():
                fire_gathers(c + 1, q)

            wait_gathers(c, p)

            @pl.when(c >= 2)
            def _():
                out_copy(c - 2, p).wait()

            @pl.loop(0, D // L)
            def _cols(j):
                sl = pl.ds(j * L, L)
                for i in range(C):
                    acc = irows[p, i, sl] + orows[p, i, sl]
                    for f in range(F):
                        acc = acc + arows[p, i * F + f, sl]
                    out_v[p, i, sl] = acc

            out_copy(c, p).start()

    out_copy(NCHUNK - 2, 0).wait()
    out_copy(NCHUNK - 1, 1).wait()


def kernel(x, in_degree, out_degree, atom_table, in_deg_table, out_deg_table,
           graph_token):
    x3 = x.reshape(NW, NCHUNK, C * F)
    ind3 = in_degree.reshape(NW, NCHUNK, C)
    outd3 = out_degree.reshape(NW, NCHUNK, C)
    out = _sc_body(x3, ind3, outd3, atom_table, in_deg_table, out_deg_table,
                   graph_token)
    return out.reshape(B, N + 1, D)


# speedup vs baseline: 1.1553x; 1.0366x over previous
import functools

import jax
import jax.numpy as jnp
from jax import lax
from jax.experimental import pallas as pl
from jax.experimental.pallas import tpu as pltpu
from jax.experimental.pallas import tpu_sc as plsc

B, N, F, D = 64, 256, 9, 768
NC, NS, L = 2, 16, 16
NW = NC * NS
R = B * N
RPW = R // NW
C = 16
NCHUNK = RPW // C
NSLOT = 4

_mesh = plsc.VectorSubcoreMesh(core_axis_name="c", subcore_axis_name="s")


@functools.partial(
    pl.kernel,
    out_type=jax.ShapeDtypeStruct((B, N + 1, D), jnp.float32),
    mesh=_mesh,
    compiler_params=pltpu.CompilerParams(use_tc_tiling_on_sc=False),
    scratch_types=[
        pltpu.VMEM((NCHUNK, F, C), jnp.int32),
        pltpu.VMEM((NCHUNK, C), jnp.int32),
        pltpu.VMEM((NCHUNK, C), jnp.int32),
        pltpu.VMEM((NSLOT, C, D), jnp.float32),
        pltpu.VMEM((1, D), jnp.float32),
        [pltpu.SemaphoreType.DMA] * NSLOT,
        [pltpu.SemaphoreType.DMA] * NSLOT,
    ],
)
def _sc_body(x_hbm, ind_hbm, outd_hbm, atab, itab, otab, tok, out_hbm,
             x_v, ind_v, outd_v, acc, tok_v, semg, semo):
    w = lax.axis_index("s") * NC + lax.axis_index("c")

    pltpu.sync_copy(x_hbm.at[w], x_v)
    pltpu.sync_copy(ind_hbm.at[w], ind_v)
    pltpu.sync_copy(outd_hbm.at[w], outd_v)
    pltpu.sync_copy(tok, tok_v)
    b0 = w * (RPW // N)
    for k in range(RPW // N):
        pltpu.sync_copy(tok_v, out_hbm.at[b0 + k, pl.ds(0, 1)])

    def zero_slot(p):
        z = jnp.zeros((L,), jnp.float32)

        @pl.loop(0, D // L)
        def _z(j):
            sl = pl.ds(j * L, L)
            for i in range(C):
                acc[p, i, sl] = z

    def gathers(c, p):
        copies = [pltpu.make_async_copy(atab.at[x_v.at[c, f]], acc.at[p],
                                        semg[p]) for f in range(F)]
        copies.append(pltpu.make_async_copy(itab.at[ind_v.at[c]], acc.at[p],
                                            semg[p]))
        copies.append(pltpu.make_async_copy(otab.at[outd_v.at[c]], acc.at[p],
                                            semg[p]))
        return copies

    def fire_gathers(c, p):
        for f in range(F):
            pltpu.async_copy(atab.at[x_v.at[c, f]], acc.at[p], semg[p],
                             add=True)
        pltpu.async_copy(itab.at[ind_v.at[c]], acc.at[p], semg[p], add=True)
        pltpu.async_copy(otab.at[outd_v.at[c]], acc.at[p], semg[p], add=True)

    def wait_gathers(c, p):
        for cp in gathers(c, p):
            cp.wait()

    def out_copy(c, p):
        r0 = w * RPW + c * C
        return pltpu.make_async_copy(
            acc.at[p], out_hbm.at[r0 // N, pl.ds(r0 % N + 1, C)], semo[p])

    zero_slot(0)
    fire_gathers(0, 0)

    @pl.loop(0, NCHUNK, step=NSLOT)
    def _cs(c0):
        for p in range(NSLOT):
            c = c0 + p
            p1 = (p + 1) % NSLOT

            @pl.when(c + 1 < NCHUNK)
            def _():
                @pl.when(c + 1 >= NSLOT)
                def _():
                    out_copy(c + 1 - NSLOT, p1).wait()

                zero_slot(p1)
                fire_gathers(c + 1, p1)

            wait_gathers(c, p)
            out_copy(c, p).start()

    for k in range(NSLOT):
        out_copy(NCHUNK - NSLOT + k, k).wait()


def kernel(x, in_degree, out_degree, atom_table, in_deg_table, out_deg_table,
           graph_token):
    x4 = x.reshape(NW, NCHUNK, C, F).transpose(0, 1, 3, 2)
    ind3 = in_degree.reshape(NW, NCHUNK, C)
    outd3 = out_degree.reshape(NW, NCHUNK, C)
    return _sc_body(x4, ind3, outd3, atom_table, in_deg_table, out_deg_table,
                    graph_token)


# speedup vs baseline: 1.3447x; 1.1640x over previous
import functools

import jax
import jax.numpy as jnp
from jax import lax
from jax.experimental import pallas as pl
from jax.experimental.pallas import tpu as pltpu
from jax.experimental.pallas import tpu_sc as plsc

B, N, F, D = 64, 256, 9, 768
NC, NS, L = 2, 16, 16
NW = NC * NS
NPW = N // NW
C = 32
HB = B // C
NCHUNK = NPW * HB
NSLOT = 4
OUT_ROWS = (N + 1) * B

_mesh = plsc.VectorSubcoreMesh(core_axis_name="c", subcore_axis_name="s")


@functools.partial(
    pl.kernel,
    out_type=jax.ShapeDtypeStruct((OUT_ROWS, D), jnp.float32),
    mesh=_mesh,
    compiler_params=pltpu.CompilerParams(use_tc_tiling_on_sc=False),
    scratch_types=[
        pltpu.VMEM((NPW, F, HB, C), jnp.int32),
        pltpu.VMEM((NPW, HB, C), jnp.int32),
        pltpu.VMEM((NPW, HB, C), jnp.int32),
        pltpu.VMEM((NSLOT, C, D), jnp.float32),
        pltpu.VMEM((2, D), jnp.float32),
        [pltpu.SemaphoreType.DMA] * NSLOT,
        [pltpu.SemaphoreType.DMA] * NSLOT,
    ],
)
def _sc_body(x_hbm, ind_hbm, outd_hbm, atab, itab, otab, tok, out_hbm,
             x_v, ind_v, outd_v, acc, tok_v, semg, semo):
    w = lax.axis_index("s") * NC + lax.axis_index("c")

    pltpu.sync_copy(x_hbm.at[w], x_v)
    pltpu.sync_copy(ind_hbm.at[w], ind_v)
    pltpu.sync_copy(outd_hbm.at[w], outd_v)
    pltpu.sync_copy(tok, tok_v.at[pl.ds(0, 1)])
    pltpu.sync_copy(tok, tok_v.at[pl.ds(1, 1)])
    pltpu.sync_copy(tok_v, out_hbm.at[pl.ds(2 * w, 2)])

    def zero_slot(p):
        z = jnp.zeros((L,), jnp.float32)

        @pl.loop(0, D // L)
        def _z(j):
            sl = pl.ds(j * L, L)
            for i in range(C):
                acc[p, i, sl] = z

    def gathers(c, p):
        k = c // HB
        h = c % HB
        copies = [pltpu.make_async_copy(atab.at[x_v.at[k, f, h]], acc.at[p],
                                        semg[p]) for f in range(F)]
        copies.append(pltpu.make_async_copy(itab.at[ind_v.at[k, h]], acc.at[p],
                                            semg[p]))
        copies.append(pltpu.make_async_copy(otab.at[outd_v.at[k, h]],
                                            acc.at[p], semg[p]))
        return copies

    def fire_gathers(c, p):
        k = c // HB
        h = c % HB
        for f in range(F):
            pltpu.async_copy(atab.at[x_v.at[k, f, h]], acc.at[p], semg[p],
                             add=True)
        pltpu.async_copy(itab.at[ind_v.at[k, h]], acc.at[p], semg[p],
                         add=True)
        pltpu.async_copy(otab.at[outd_v.at[k, h]], acc.at[p], semg[p],
                         add=True)

    def wait_gathers(c, p):
        for cp in gathers(c, p):
            cp.wait()

    def out_copy(c, p):
        row0 = (w * NPW + c // HB + 1) * B + (c % HB) * C
        return pltpu.make_async_copy(
            acc.at[p], out_hbm.at[pl.ds(row0, C)], semo[p])

    zero_slot(0)
    fire_gathers(0, 0)

    @pl.loop(0, NCHUNK, step=NSLOT)
    def _cs(c0):
        for p in range(NSLOT):
            c = c0 + p
            p1 = (p + 1) % NSLOT

            @pl.when(c + 1 < NCHUNK)
            def _():
                @pl.when(c + 1 >= NSLOT)
                def _():
                    out_copy(c + 1 - NSLOT, p1).wait()

                zero_slot(p1)
                fire_gathers(c + 1, p1)

            wait_gathers(c, p)
            out_copy(c, p).start()

    for k in range(NSLOT):
        out_copy(NCHUNK - NSLOT + k, k).wait()


def kernel(x, in_degree, out_degree, atom_table, in_deg_table, out_deg_table,
           graph_token):
    x5 = x.transpose(1, 2, 0).reshape(NW, NPW, F, HB, C)
    ind4 = in_degree.transpose(1, 0).reshape(NW, NPW, HB, C)
    outd4 = out_degree.transpose(1, 0).reshape(NW, NPW, HB, C)
    out = _sc_body(x5, ind4, outd4, atom_table, in_deg_table, out_deg_table,
                   graph_token)
    return out.reshape(N + 1, B, D).transpose(1, 0, 2)
